# baseline (speedup 1.0000x reference)
import functools

import jax
import jax.numpy as jnp
from jax import lax
from jax.experimental import pallas as pl
from jax.experimental.pallas import tpu as pltpu
from jax.experimental.pallas import tpu_sc as plsc

_LANES = 16


@functools.lru_cache(maxsize=None)
def _build_sc_embed(n_rows: int, vocab: int, hidden: int):
    info = plsc.get_sparse_core_info()
    nc, ns = info.num_cores, info.num_subcores
    nw = nc * ns
    assert n_rows % (nw * _LANES) == 0
    rows_per_w = n_rows // nw

    mesh = plsc.VectorSubcoreMesh(core_axis_name="c", subcore_axis_name="s")

    @functools.partial(
        pl.kernel,
        out_type=jax.ShapeDtypeStruct((n_rows, hidden), jnp.float32),
        mesh=mesh,
        scratch_types=[
            pltpu.VMEM((vocab, hidden), jnp.float32),
            pltpu.VMEM((rows_per_w,), jnp.int32),
            pltpu.SemaphoreType.DMA,
            pltpu.SemaphoreType.DMA,
        ],
    )
    def embed(x_hbm, table_hbm, out_hbm, table_v, idx_v, sem_in, sem_out):
        wid = lax.axis_index("s") * nc + lax.axis_index("c")
        base = wid * rows_per_w
        tab_cp = pltpu.make_async_copy(table_hbm, table_v, sem_in)
        idx_cp = pltpu.make_async_copy(
            x_hbm.at[pl.ds(base, rows_per_w)], idx_v, sem_in
        )
        tab_cp.start()
        idx_cp.start()
        tab_cp.wait()
        idx_cp.wait()

        def body(g, carry):
            row0 = g * _LANES
            xv = idx_v[pl.ds(row0, _LANES)]
            for j in range(_LANES):
                pltpu.make_async_copy(
                    table_v.at[xv[j]], out_hbm.at[base + row0 + j], sem_out
                ).start()
            return carry

        lax.fori_loop(0, rows_per_w // _LANES, body, 0)

        mine = out_hbm.at[pl.ds(base, rows_per_w)]
        pltpu.make_async_copy(mine, mine, sem_out).wait()

    return embed


def kernel(x, table):
    b, s = x.shape
    n = b * s
    xf = x.reshape(n).astype(jnp.int32)
    out_flat = _build_sc_embed(n, table.shape[0], table.shape[1])(xf, table)
    return out_flat.reshape(b, s, table.shape[1])


# speedup vs baseline: 1.0031x; 1.0031x over previous
import functools

import jax
import jax.numpy as jnp
from jax import lax
from jax.experimental import pallas as pl
from jax.experimental.pallas import tpu as pltpu
from jax.experimental.pallas import tpu_sc as plsc

_LANES = 16


@functools.lru_cache(maxsize=None)
def _build_sc_embed(batch: int, seq: int, vocab: int, hidden: int):
    n_rows = batch * seq
    info = plsc.get_sparse_core_info()
    nc, ns = info.num_cores, info.num_subcores
    nw = nc * ns
    assert n_rows % (nw * _LANES) == 0 and seq % (nw // batch) == 0
    rows_per_w = n_rows // nw
    w_per_b = nw // batch

    mesh = plsc.VectorSubcoreMesh(core_axis_name="c", subcore_axis_name="s")

    @functools.partial(
        pl.kernel,
        out_type=jax.ShapeDtypeStruct((n_rows, hidden), jnp.float32),
        mesh=mesh,
        scratch_types=[
            pltpu.VMEM((vocab, hidden), jnp.float32),
            pltpu.VMEM((rows_per_w,), jnp.int32),
            pltpu.SemaphoreType.DMA,
            pltpu.SemaphoreType.DMA,
        ],
    )
    def embed(x_hbm, table_hbm, out_hbm, table_v, idx_v, sem_in, sem_out):
        wid = lax.axis_index("s") * nc + lax.axis_index("c")
        base = wid * rows_per_w
        tab_cp = pltpu.make_async_copy(table_hbm, table_v, sem_in)
        idx_cp = pltpu.make_async_copy(
            x_hbm.at[wid // w_per_b, pl.ds((wid % w_per_b) * rows_per_w, rows_per_w)],
            idx_v,
            sem_in,
        )
        tab_cp.start()
        idx_cp.start()
        tab_cp.wait()
        idx_cp.wait()

        def body(g, carry):
            row0 = g * _LANES
            xv = idx_v[pl.ds(row0, _LANES)]
            for j in range(_LANES):
                pltpu.make_async_copy(
                    table_v.at[xv[j]], out_hbm.at[base + row0 + j], sem_out
                ).start()
            return carry

        lax.fori_loop(0, rows_per_w // _LANES, body, 0)

        mine = out_hbm.at[pl.ds(base, rows_per_w)]
        pltpu.make_async_copy(mine, mine, sem_out).wait()

    return embed


def kernel(x, table):
    b, s = x.shape
    xf = x.astype(jnp.int32)
    out_flat = _build_sc_embed(b, s, table.shape[0], table.shape[1])(xf, table)
    return out_flat.reshape(b, s, table.shape[1])


# speedup vs baseline: 1.0265x; 1.0234x over previous
import functools

import jax
import jax.numpy as jnp
from jax import lax
from jax.experimental import pallas as pl
from jax.experimental.pallas import tpu as pltpu
from jax.experimental.pallas import tpu_sc as plsc

_LANES = 16
_SP_ROWS = (0, 3, 6, 9, 12, 15)


@functools.lru_cache(maxsize=None)
def _build_sc_embed(batch: int, seq: int, vocab: int, hidden: int):
    n_rows = batch * seq
    info = plsc.get_sparse_core_info()
    nc, ns = info.num_cores, info.num_subcores
    nw = nc * ns
    assert n_rows % (nw * _LANES) == 0 and (nw % batch == 0)
    rows_per_w = n_rows // nw
    w_per_b = nw // batch
    n_sp = len(_SP_ROWS)

    mesh = plsc.VectorSubcoreMesh(core_axis_name="c", subcore_axis_name="s")

    @functools.partial(
        pl.kernel,
        out_type=jax.ShapeDtypeStruct((n_rows, hidden), jnp.float32),
        mesh=mesh,
        scratch_types=[
            pltpu.VMEM((vocab, hidden), jnp.float32),
            pltpu.VMEM((rows_per_w,), jnp.int32),
            pltpu.VMEM_SHARED((vocab, hidden), jnp.float32),
            pltpu.SemaphoreType.DMA,
            pltpu.SemaphoreType.DMA,
            pltpu.SemaphoreType.DMA,
        ],
    )
    def embed(
        x_hbm, table_hbm, out_hbm, table_v, idx_v, table_sp, sem_in, sem_t, sem_s
    ):
        cid = lax.axis_index("c")
        sid = lax.axis_index("s")
        wid = sid * nc + cid
        base = wid * rows_per_w
        tab_cp = pltpu.make_async_copy(table_hbm, table_v, sem_in)
        idx_cp = pltpu.make_async_copy(
            x_hbm.at[wid // w_per_b, pl.ds((wid % w_per_b) * rows_per_w, rows_per_w)],
            idx_v,
            sem_in,
        )
        tab_cp.start()
        idx_cp.start()
        tab_cp.wait()
        idx_cp.wait()

        @pl.when(sid == 0)
        def _stage_shared():
            pltpu.sync_copy(table_v, table_sp)

        plsc.subcore_barrier()

        def body(g, carry):
            row0 = g * _LANES
            xv = idx_v[pl.ds(row0, _LANES)]
            for j in range(_LANES):
                if j in _SP_ROWS:
                    src, sem = table_sp.at[xv[j]], sem_s
                else:
                    src, sem = table_v.at[xv[j]], sem_t
                pltpu.make_async_copy(src, out_hbm.at[base + row0 + j], sem).start()
            return carry

        lax.fori_loop(0, rows_per_w // _LANES, body, 0)

        sp_rows_total = (rows_per_w // _LANES) * n_sp
        pltpu.make_async_copy(
            out_hbm.at[pl.ds(base, sp_rows_total)],
            out_hbm.at[pl.ds(base, sp_rows_total)],
            sem_s,
        ).wait()
        pltpu.make_async_copy(
            out_hbm.at[pl.ds(base, rows_per_w - sp_rows_total)],
            out_hbm.at[pl.ds(base, rows_per_w - sp_rows_total)],
            sem_t,
        ).wait()

    return embed


def kernel(x, table):
    b, s = x.shape
    xf = x.astype(jnp.int32)
    out_flat = _build_sc_embed(b, s, table.shape[0], table.shape[1])(xf, table)
    return out_flat.reshape(b, s, table.shape[1])
